# baseline (speedup 1.0000x reference)
import functools

import jax
import jax.numpy as jnp
from jax import lax
from jax.experimental import pallas as pl
from jax.experimental.pallas import tpu as pltpu
from jax.experimental.pallas import tpu_sc as plsc

NC = 2
NS = 16
NW = NC * NS
LANE = 128


def _mesh():
    return plsc.VectorSubcoreMesh(
        core_axis_name="c", subcore_axis_name="s", num_cores=NC, num_subcores=NS
    )


def _fill(ref, n, value):
    def body(i, _):
        ref[pl.ds(i * 16, 16)] = jnp.full((16,), value, jnp.float32)
        return 0
    lax.fori_loop(0, n // 16, body, 0)


def _sc_degrees(src, dst, n_pad):
    E = src.shape[0]
    CH = 128
    assert E % NW == 0
    per_tile = E // NW
    full = per_tile // CH
    tail = per_tile % CH
    pairs, odd = divmod(full, 2)
    assert tail % 16 == 0 and per_tile % 8 == 0
    rpt = n_pad // NS

    def body(s_hbm, d_hbm, out_hbm, sbA, dbA, sbB, dbB, sbt, dbt, ones_v,
             onest_v, zer_v, din_sp, dout_sp, semA, semB, semC, semD,
             semS1, semS2, semS3, semS4):
        c = lax.axis_index("c")
        s = lax.axis_index("s")
        wid = c * NS + s
        base0 = wid * per_tile

        _fill(zer_v, rpt, 0.0)
        _fill(ones_v, CH, 1.0)
        if tail:
            _fill(onest_v, tail, 1.0)
        pltpu.sync_copy(zer_v, din_sp.at[pl.ds(s * rpt, rpt)])
        pltpu.sync_copy(zer_v, dout_sp.at[pl.ds(s * rpt, rpt)])
        plsc.subcore_barrier()

        def pbody(i, _):
            b0 = base0 + (2 * i) * CH
            b1 = b0 + CH
            dA = pltpu.async_copy(s_hbm.at[pl.ds(b0, CH)], sbA, semA)
            dB = pltpu.async_copy(d_hbm.at[pl.ds(b0, CH)], dbA, semB)
            dC = pltpu.async_copy(s_hbm.at[pl.ds(b1, CH)], sbB, semC)
            dD = pltpu.async_copy(d_hbm.at[pl.ds(b1, CH)], dbB, semD)
            dA.wait()
            s1 = pltpu.async_copy(ones_v, dout_sp.at[sbA], semS1, add=True)
            dB.wait()
            s2 = pltpu.async_copy(ones_v, din_sp.at[dbA], semS2, add=True)
            dC.wait()
            s3 = pltpu.async_copy(ones_v, dout_sp.at[sbB], semS3, add=True)
            dD.wait()
            s4 = pltpu.async_copy(ones_v, din_sp.at[dbB], semS4, add=True)
            s1.wait()
            s2.wait()
            s3.wait()
            s4.wait()
            return 0

        lax.fori_loop(0, pairs, pbody, 0)

        if odd:
            b0 = base0 + (pairs * 2) * CH
            pltpu.sync_copy(s_hbm.at[pl.ds(b0, CH)], sbA)
            pltpu.sync_copy(ones_v, dout_sp.at[sbA], add=True)
            pltpu.sync_copy(d_hbm.at[pl.ds(b0, CH)], dbA)
            pltpu.sync_copy(ones_v, din_sp.at[dbA], add=True)

        if tail:
            bt = base0 + full * CH
            pltpu.sync_copy(s_hbm.at[pl.ds(bt, tail)], sbt)
            pltpu.sync_copy(onest_v, dout_sp.at[sbt], add=True)
            pltpu.sync_copy(d_hbm.at[pl.ds(bt, tail)], dbt)
            pltpu.sync_copy(onest_v, din_sp.at[dbt], add=True)

        plsc.subcore_barrier()

        pltpu.sync_copy(din_sp.at[pl.ds(s * rpt, rpt)],
                        out_hbm.at[c, 0, pl.ds(s * rpt, rpt)])
        pltpu.sync_copy(dout_sp.at[pl.ds(s * rpt, rpt)],
                        out_hbm.at[c, 1, pl.ds(s * rpt, rpt)])

    f = pl.kernel(
        body,
        out_type=jax.ShapeDtypeStruct((NC, 2, n_pad), jnp.float32),
        mesh=_mesh(),
        scratch_types=[
            pltpu.VMEM((CH,), jnp.int32),
            pltpu.VMEM((CH,), jnp.int32),
            pltpu.VMEM((CH,), jnp.int32),
            pltpu.VMEM((CH,), jnp.int32),
            pltpu.VMEM((max(tail, 16),), jnp.int32),
            pltpu.VMEM((max(tail, 16),), jnp.int32),
            pltpu.VMEM((CH,), jnp.float32),
            pltpu.VMEM((max(tail, 16),), jnp.float32),
            pltpu.VMEM((rpt,), jnp.float32),
            pltpu.VMEM_SHARED((n_pad,), jnp.float32),
            pltpu.VMEM_SHARED((n_pad,), jnp.float32),
            pltpu.SemaphoreType.DMA,
            pltpu.SemaphoreType.DMA,
            pltpu.SemaphoreType.DMA,
            pltpu.SemaphoreType.DMA,
            pltpu.SemaphoreType.DMA,
            pltpu.SemaphoreType.DMA,
            pltpu.SemaphoreType.DMA,
            pltpu.SemaphoreType.DMA,
        ],
        compiler_params=pltpu.CompilerParams(use_tc_tiling_on_sc=False),
    )
    return f(src, dst)


def _sc_aggregate(src, dst, p, n_pad):
    E = src.shape[0]
    H = p.shape[1]
    CH = 128
    assert E % NW == 0
    per_tile = E // NW
    full = per_tile // CH
    tail = per_tile % CH
    triples, rem = divmod(full, 3)
    assert tail % 8 == 0 and per_tile % 8 == 0
    rpt = n_pad // NS
    ZR = 64
    assert rpt % ZR == 0

    def body(s_hbm, d_hbm, p_hbm, out_hbm, sbig, dbA, dbB, dbC, dbt, rowsA, rowsB,
             rowsC, rowst, zer_v, acc_sp, semDA, semDB, semDC, semGA, semGB,
             semGC, semSA, semSB, semSC):
        c = lax.axis_index("c")
        s = lax.axis_index("s")
        wid = c * NS + s
        base0 = wid * per_tile

        def zfill(i, _):
            zer_v[i, pl.ds(0, 16)] = jnp.zeros((16,), jnp.float32)
            zer_v[i, pl.ds(16, 16)] = jnp.zeros((16,), jnp.float32)
            zer_v[i, pl.ds(32, 16)] = jnp.zeros((16,), jnp.float32)
            zer_v[i, pl.ds(48, 16)] = jnp.zeros((16,), jnp.float32)
            return 0
        lax.fori_loop(0, ZR, zfill, 0)

        def zcopy(i, _):
            pltpu.sync_copy(zer_v, acc_sp.at[pl.ds(s * rpt + i * ZR, ZR)])
            return 0
        lax.fori_loop(0, rpt // ZR, zcopy, 0)

        pltpu.sync_copy(s_hbm.at[pl.ds(base0, per_tile)], sbig)
        plsc.subcore_barrier()

        bufs = ((dbA, rowsA, semDA, semGA, semSA),
                (dbB, rowsB, semDB, semGB, semSB),
                (dbC, rowsC, semDC, semGC, semSC))

        def tbody(i, _):
            descs = []
            for k, (db, rows, semD, semG, semS) in enumerate(bufs):
                b0 = (3 * i + k) * CH
                @pl.when(i > 0)
                def _(db=db, rows=rows, semS=semS):
                    pltpu.make_async_copy(rows, acc_sp.at[db], semS).wait()
                descs.append((
                    pltpu.async_copy(d_hbm.at[pl.ds(base0 + b0, CH)], db, semD),
                    pltpu.async_copy(p_hbm.at[sbig.at[pl.ds(b0, CH)]], rows, semG),
                ))
            for (d, g), (db, rows, _, _, semS) in zip(descs, bufs):
                d.wait()
                g.wait()
                pltpu.async_copy(rows, acc_sp.at[db], semS, add=True)
            return 0

        lax.fori_loop(0, triples, tbody, 0)
        if triples > 0:
            for db, rows, _, _, semS in bufs:
                pltpu.make_async_copy(rows, acc_sp.at[db], semS).wait()

        for r in range(rem):
            b0 = (triples * 3 + r) * CH
            pltpu.sync_copy(d_hbm.at[pl.ds(base0 + b0, CH)], dbA)
            pltpu.async_copy(p_hbm.at[sbig.at[pl.ds(b0, CH)]], rowsA, semGA).wait()
            pltpu.sync_copy(rowsA, acc_sp.at[dbA], add=True)

        if tail:
            bt = full * CH
            pltpu.sync_copy(d_hbm.at[pl.ds(base0 + bt, tail)], dbt)
            pltpu.async_copy(p_hbm.at[sbig.at[pl.ds(bt, tail)]], rowst, semGA).wait()
            pltpu.sync_copy(rowst, acc_sp.at[dbt], add=True)

        plsc.subcore_barrier()
        pltpu.sync_copy(acc_sp.at[pl.ds(s * rpt, rpt)],
                        out_hbm.at[pl.ds(s * rpt, rpt), pl.ds(c * H, H)])

    f = pl.kernel(
        body,
        out_type=jax.ShapeDtypeStruct((n_pad, NC * H), jnp.float32),
        mesh=_mesh(),
        scratch_types=[
            pltpu.VMEM((per_tile,), jnp.int32),
            pltpu.VMEM((CH,), jnp.int32),
            pltpu.VMEM((CH,), jnp.int32),
            pltpu.VMEM((CH,), jnp.int32),
            pltpu.VMEM((max(tail, 8),), jnp.int32),
            pltpu.VMEM((CH, H), jnp.float32),
            pltpu.VMEM((CH, H), jnp.float32),
            pltpu.VMEM((CH, H), jnp.float32),
            pltpu.VMEM((max(tail, 8), H), jnp.float32),
            pltpu.VMEM((ZR, H), jnp.float32),
            pltpu.VMEM_SHARED((n_pad, H), jnp.float32),
            pltpu.SemaphoreType.DMA,
            pltpu.SemaphoreType.DMA,
            pltpu.SemaphoreType.DMA,
            pltpu.SemaphoreType.DMA,
            pltpu.SemaphoreType.DMA,
            pltpu.SemaphoreType.DMA,
            pltpu.SemaphoreType.DMA,
            pltpu.SemaphoreType.DMA,
            pltpu.SemaphoreType.DMA,
        ],
        compiler_params=pltpu.CompilerParams(use_tc_tiling_on_sc=False),
    )
    return f(src, dst, p)


def _tc_encode(text_f, vis_f, W_t, b_t, W_v, b_v, W_g0):
    N, T = text_f.shape
    V = vis_f.shape[1]
    H = W_t.shape[1]
    RB = 1000
    assert N % RB == 0

    def body(t_ref, v_ref, wt_ref, bt_ref, wv_ref, bv_ref, wg_ref, o_ref):
        ht = jnp.maximum(
            jnp.dot(t_ref[...], wt_ref[...], preferred_element_type=jnp.float32)
            + bt_ref[...], 0.0)
        hv = jnp.maximum(
            jnp.dot(v_ref[...], wv_ref[...], preferred_element_type=jnp.float32)
            + bv_ref[...], 0.0)
        h = jnp.concatenate([ht, hv], axis=1)
        o_ref[...] = jnp.dot(h, wg_ref[...], preferred_element_type=jnp.float32)

    return pl.pallas_call(
        body,
        grid=(N // RB,),
        in_specs=[
            pl.BlockSpec((RB, T), lambda i: (i, 0)),
            pl.BlockSpec((RB, V), lambda i: (i, 0)),
            pl.BlockSpec((T, H), lambda i: (0, 0)),
            pl.BlockSpec((1, H), lambda i: (0, 0)),
            pl.BlockSpec((V, H), lambda i: (0, 0)),
            pl.BlockSpec((1, H), lambda i: (0, 0)),
            pl.BlockSpec((2 * H, H), lambda i: (0, 0)),
        ],
        out_specs=pl.BlockSpec((RB, H), lambda i: (i, 0)),
        out_shape=jax.ShapeDtypeStruct((N, H), jnp.float32),
    )(text_f, vis_f, W_t, b_t, W_v, b_v, W_g0)


def _tc_prescale(z, deg, N, n_pad):
    H = z.shape[1]

    def body(z_ref, dg_ref, o_ref, iv_ref):
        d = dg_ref[0] + dg_ref[1]
        inv = 1.0 / jnp.sqrt(jnp.maximum(d, 1.0))
        invt = jnp.transpose(inv, (1, 0))
        iv_ref[...] = invt
        o_ref[...] = z_ref[...] * invt[:N, 1:2]

    return pl.pallas_call(
        body,
        out_shape=(
            jax.ShapeDtypeStruct((N, H), jnp.float32),
            jax.ShapeDtypeStruct((n_pad, 2), jnp.float32),
        ),
    )(z, deg)


def _tc_mid(agg, invs, b_g0, W_g1, N):
    H = W_g1.shape[0]
    RB = 1000
    assert N % RB == 0

    def body(a_ref, iv_ref, b_ref, w_ref, o_ref):
        a = a_ref[:, :H] + a_ref[:, H:]
        iv = iv_ref[...]
        h = jnp.maximum(a * iv[:, 0:1] + b_ref[...], 0.0)
        z = jnp.dot(h, w_ref[...], preferred_element_type=jnp.float32)
        o_ref[...] = z * iv[:, 1:2]

    return pl.pallas_call(
        body,
        grid=(N // RB,),
        in_specs=[
            pl.BlockSpec((RB, NC * H), lambda i: (i, 0)),
            pl.BlockSpec((RB, 2), lambda i: (i, 0)),
            pl.BlockSpec((1, H), lambda i: (0, 0)),
            pl.BlockSpec((H, H), lambda i: (0, 0)),
        ],
        out_specs=pl.BlockSpec((RB, H), lambda i: (i, 0)),
        out_shape=jax.ShapeDtypeStruct((N, H), jnp.float32),
    )(agg, invs, b_g0, W_g1)


def _tc_head(agg, invs, b_g1, W_head, b_head, N):
    H, C = W_head.shape
    RB = 1000
    assert N % RB == 0

    def body(a_ref, iv_ref, b_ref, w_ref, bh_ref, o_ref):
        a = a_ref[:, :H] + a_ref[:, H:]
        iv = iv_ref[...]
        h = jnp.maximum(a * iv[:, 0:1] + b_ref[...], 0.0)
        o_ref[...] = (
            jnp.dot(h, w_ref[...], preferred_element_type=jnp.float32) + bh_ref[...]
        )

    return pl.pallas_call(
        body,
        grid=(N // RB,),
        in_specs=[
            pl.BlockSpec((RB, NC * H), lambda i: (i, 0)),
            pl.BlockSpec((RB, 2), lambda i: (i, 0)),
            pl.BlockSpec((1, H), lambda i: (0, 0)),
            pl.BlockSpec((H, C), lambda i: (0, 0)),
            pl.BlockSpec((1, C), lambda i: (0, 0)),
        ],
        out_specs=pl.BlockSpec((RB, C), lambda i: (i, 0)),
        out_shape=jax.ShapeDtypeStruct((N, C), jnp.float32),
    )(agg, invs, b_g1, W_head, b_head)


def kernel(edge_index, text_f, vis_f, W_t, b_t, W_v, b_v, W_g0, b_g0, W_g1, b_g1,
           W_head, b_head):
    N = text_f.shape[0]
    E = edge_index.shape[1]
    n_pad = -(-N // (NS * 64)) * (NS * 64)

    src = edge_index[0]
    dst = edge_index[1]
    deg = _sc_degrees(src, dst, n_pad)
    z0 = _tc_encode(text_f, vis_f, W_t, b_t.reshape(1, -1), W_v,
                    b_v.reshape(1, -1), W_g0)
    p0, invs = _tc_prescale(z0, deg, N, n_pad)
    agg0 = _sc_aggregate(src, dst, p0, n_pad)
    p1 = _tc_mid(agg0, invs, b_g0.reshape(1, -1), W_g1, N)
    agg1 = _sc_aggregate(src, dst, p1, n_pad)
    out = _tc_head(agg1, invs, b_g1.reshape(1, -1), W_head, b_head.reshape(1, -1), N)
    return out


# speedup vs baseline: 1.0461x; 1.0461x over previous
import functools

import jax
import jax.numpy as jnp
from jax import lax
from jax.experimental import pallas as pl
from jax.experimental.pallas import tpu as pltpu
from jax.experimental.pallas import tpu_sc as plsc

NC = 2
NS = 16
NW = NC * NS
LANE = 128


def _mesh():
    return plsc.VectorSubcoreMesh(
        core_axis_name="c", subcore_axis_name="s", num_cores=NC, num_subcores=NS
    )


def _fill(ref, n, value):
    def body(i, _):
        ref[pl.ds(i * 16, 16)] = jnp.full((16,), value, jnp.float32)
        return 0
    lax.fori_loop(0, n // 16, body, 0)


def _sc_degrees(eflat, n_pad):
    E = eflat.shape[1]
    CH = 128
    assert E % NW == 0
    per_tile = E // NW
    full = per_tile // CH
    tail = per_tile % CH
    pairs, odd = divmod(full, 2)
    assert tail % 16 == 0 and per_tile % 8 == 0
    rpt = n_pad // NS

    def body(e_hbm, out_hbm, sbA, dbA, sbB, dbB, sbt, dbt, ones_v,
             onest_v, zer_v, din_sp, dout_sp, semA, semB, semC, semD,
             semS1, semS2, semS3, semS4):
        c = lax.axis_index("c")
        s = lax.axis_index("s")
        wid = c * NS + s
        base0 = wid * per_tile

        _fill(zer_v, rpt, 0.0)
        _fill(ones_v, CH, 1.0)
        if tail:
            _fill(onest_v, tail, 1.0)
        pltpu.sync_copy(zer_v, din_sp.at[pl.ds(s * rpt, rpt)])
        pltpu.sync_copy(zer_v, dout_sp.at[pl.ds(s * rpt, rpt)])
        plsc.subcore_barrier()

        def pbody(i, _):
            b0 = base0 + (2 * i) * CH
            b1 = b0 + CH
            dA = pltpu.async_copy(e_hbm.at[0, pl.ds(b0, CH)], sbA, semA)
            dB = pltpu.async_copy(e_hbm.at[1, pl.ds(b0, CH)], dbA, semB)
            dC = pltpu.async_copy(e_hbm.at[0, pl.ds(b1, CH)], sbB, semC)
            dD = pltpu.async_copy(e_hbm.at[1, pl.ds(b1, CH)], dbB, semD)
            dA.wait()
            s1 = pltpu.async_copy(ones_v, dout_sp.at[sbA], semS1, add=True)
            dB.wait()
            s2 = pltpu.async_copy(ones_v, din_sp.at[dbA], semS2, add=True)
            dC.wait()
            s3 = pltpu.async_copy(ones_v, dout_sp.at[sbB], semS3, add=True)
            dD.wait()
            s4 = pltpu.async_copy(ones_v, din_sp.at[dbB], semS4, add=True)
            s1.wait()
            s2.wait()
            s3.wait()
            s4.wait()
            return 0

        lax.fori_loop(0, pairs, pbody, 0)

        if odd:
            b0 = base0 + (pairs * 2) * CH
            pltpu.sync_copy(e_hbm.at[0, pl.ds(b0, CH)], sbA)
            pltpu.sync_copy(ones_v, dout_sp.at[sbA], add=True)
            pltpu.sync_copy(e_hbm.at[1, pl.ds(b0, CH)], dbA)
            pltpu.sync_copy(ones_v, din_sp.at[dbA], add=True)

        if tail:
            bt = base0 + full * CH
            pltpu.sync_copy(e_hbm.at[0, pl.ds(bt, tail)], sbt)
            pltpu.sync_copy(onest_v, dout_sp.at[sbt], add=True)
            pltpu.sync_copy(e_hbm.at[1, pl.ds(bt, tail)], dbt)
            pltpu.sync_copy(onest_v, din_sp.at[dbt], add=True)

        plsc.subcore_barrier()

        pltpu.sync_copy(din_sp.at[pl.ds(s * rpt, rpt)],
                        out_hbm.at[c, 0, pl.ds(s * rpt, rpt)])
        pltpu.sync_copy(dout_sp.at[pl.ds(s * rpt, rpt)],
                        out_hbm.at[c, 1, pl.ds(s * rpt, rpt)])

    f = pl.kernel(
        body,
        out_type=jax.ShapeDtypeStruct((NC, 2, n_pad), jnp.float32),
        mesh=_mesh(),
        scratch_types=[
            pltpu.VMEM((CH,), jnp.int32),
            pltpu.VMEM((CH,), jnp.int32),
            pltpu.VMEM((CH,), jnp.int32),
            pltpu.VMEM((CH,), jnp.int32),
            pltpu.VMEM((max(tail, 16),), jnp.int32),
            pltpu.VMEM((max(tail, 16),), jnp.int32),
            pltpu.VMEM((CH,), jnp.float32),
            pltpu.VMEM((max(tail, 16),), jnp.float32),
            pltpu.VMEM((rpt,), jnp.float32),
            pltpu.VMEM_SHARED((n_pad,), jnp.float32),
            pltpu.VMEM_SHARED((n_pad,), jnp.float32),
            pltpu.SemaphoreType.DMA,
            pltpu.SemaphoreType.DMA,
            pltpu.SemaphoreType.DMA,
            pltpu.SemaphoreType.DMA,
            pltpu.SemaphoreType.DMA,
            pltpu.SemaphoreType.DMA,
            pltpu.SemaphoreType.DMA,
            pltpu.SemaphoreType.DMA,
        ],
        compiler_params=pltpu.CompilerParams(use_tc_tiling_on_sc=False),
    )
    return f(eflat)


def _sc_aggregate(eflat, p, n_pad):
    E = eflat.shape[1]
    H = p.shape[1]
    CH = 128
    assert E % NW == 0
    per_tile = E // NW
    full = per_tile // CH
    tail = per_tile % CH
    triples, rem = divmod(full, 3)
    assert tail % 8 == 0 and per_tile % 8 == 0
    rpt = n_pad // NS
    ZR = 64
    assert rpt % ZR == 0

    def body(e_hbm, p_hbm, out_hbm, sbig, dbA, dbB, dbC, dbt, rowsA, rowsB,
             rowsC, rowst, zer_v, acc_sp, semDA, semDB, semDC, semGA, semGB,
             semGC, semSA, semSB, semSC):
        c = lax.axis_index("c")
        s = lax.axis_index("s")
        wid = c * NS + s
        base0 = wid * per_tile

        def zfill(i, _):
            zer_v[i, pl.ds(0, 16)] = jnp.zeros((16,), jnp.float32)
            zer_v[i, pl.ds(16, 16)] = jnp.zeros((16,), jnp.float32)
            zer_v[i, pl.ds(32, 16)] = jnp.zeros((16,), jnp.float32)
            zer_v[i, pl.ds(48, 16)] = jnp.zeros((16,), jnp.float32)
            return 0
        lax.fori_loop(0, ZR, zfill, 0)

        def zcopy(i, _):
            pltpu.sync_copy(zer_v, acc_sp.at[pl.ds(s * rpt + i * ZR, ZR)])
            return 0
        lax.fori_loop(0, rpt // ZR, zcopy, 0)

        pltpu.sync_copy(e_hbm.at[0, pl.ds(base0, per_tile)], sbig)
        plsc.subcore_barrier()

        bufs = ((dbA, rowsA, semDA, semGA, semSA),
                (dbB, rowsB, semDB, semGB, semSB),
                (dbC, rowsC, semDC, semGC, semSC))

        def tbody(i, _):
            descs = []
            for k, (db, rows, semD, semG, semS) in enumerate(bufs):
                b0 = (3 * i + k) * CH
                @pl.when(i > 0)
                def _(db=db, rows=rows, semS=semS):
                    pltpu.make_async_copy(rows, acc_sp.at[db], semS).wait()
                descs.append((
                    pltpu.async_copy(e_hbm.at[1, pl.ds(base0 + b0, CH)], db, semD),
                    pltpu.async_copy(p_hbm.at[sbig.at[pl.ds(b0, CH)]], rows, semG),
                ))
            for (d, g), (db, rows, _, _, semS) in zip(descs, bufs):
                d.wait()
                g.wait()
                pltpu.async_copy(rows, acc_sp.at[db], semS, add=True)
            return 0

        lax.fori_loop(0, triples, tbody, 0)
        if triples > 0:
            for db, rows, _, _, semS in bufs:
                pltpu.make_async_copy(rows, acc_sp.at[db], semS).wait()

        for r in range(rem):
            b0 = (triples * 3 + r) * CH
            pltpu.sync_copy(e_hbm.at[1, pl.ds(base0 + b0, CH)], dbA)
            pltpu.async_copy(p_hbm.at[sbig.at[pl.ds(b0, CH)]], rowsA, semGA).wait()
            pltpu.sync_copy(rowsA, acc_sp.at[dbA], add=True)

        if tail:
            bt = full * CH
            pltpu.sync_copy(e_hbm.at[1, pl.ds(base0 + bt, tail)], dbt)
            pltpu.async_copy(p_hbm.at[sbig.at[pl.ds(bt, tail)]], rowst, semGA).wait()
            pltpu.sync_copy(rowst, acc_sp.at[dbt], add=True)

        plsc.subcore_barrier()
        pltpu.sync_copy(acc_sp.at[pl.ds(s * rpt, rpt)],
                        out_hbm.at[pl.ds(s * rpt, rpt), pl.ds(c * H, H)])

    f = pl.kernel(
        body,
        out_type=jax.ShapeDtypeStruct((n_pad, NC * H), jnp.float32),
        mesh=_mesh(),
        scratch_types=[
            pltpu.VMEM((per_tile,), jnp.int32),
            pltpu.VMEM((CH,), jnp.int32),
            pltpu.VMEM((CH,), jnp.int32),
            pltpu.VMEM((CH,), jnp.int32),
            pltpu.VMEM((max(tail, 8),), jnp.int32),
            pltpu.VMEM((CH, H), jnp.float32),
            pltpu.VMEM((CH, H), jnp.float32),
            pltpu.VMEM((CH, H), jnp.float32),
            pltpu.VMEM((max(tail, 8), H), jnp.float32),
            pltpu.VMEM((ZR, H), jnp.float32),
            pltpu.VMEM_SHARED((n_pad, H), jnp.float32),
            pltpu.SemaphoreType.DMA,
            pltpu.SemaphoreType.DMA,
            pltpu.SemaphoreType.DMA,
            pltpu.SemaphoreType.DMA,
            pltpu.SemaphoreType.DMA,
            pltpu.SemaphoreType.DMA,
            pltpu.SemaphoreType.DMA,
            pltpu.SemaphoreType.DMA,
            pltpu.SemaphoreType.DMA,
        ],
        compiler_params=pltpu.CompilerParams(use_tc_tiling_on_sc=False),
    )
    return f(eflat, p)


def _tc_encode(text_f, vis_f, W_t, b_t, W_v, b_v, W_g0):
    N, T = text_f.shape
    V = vis_f.shape[1]
    H = W_t.shape[1]
    RB = 1000
    assert N % RB == 0

    def body(t_ref, v_ref, wt_ref, bt_ref, wv_ref, bv_ref, wg_ref, o_ref):
        ht = jnp.maximum(
            jnp.dot(t_ref[...], wt_ref[...], preferred_element_type=jnp.float32)
            + bt_ref[...], 0.0)
        hv = jnp.maximum(
            jnp.dot(v_ref[...], wv_ref[...], preferred_element_type=jnp.float32)
            + bv_ref[...], 0.0)
        h = jnp.concatenate([ht, hv], axis=1)
        o_ref[...] = jnp.dot(h, wg_ref[...], preferred_element_type=jnp.float32)

    return pl.pallas_call(
        body,
        grid=(N // RB,),
        in_specs=[
            pl.BlockSpec((RB, T), lambda i: (i, 0)),
            pl.BlockSpec((RB, V), lambda i: (i, 0)),
            pl.BlockSpec((T, H), lambda i: (0, 0)),
            pl.BlockSpec((1, H), lambda i: (0, 0)),
            pl.BlockSpec((V, H), lambda i: (0, 0)),
            pl.BlockSpec((1, H), lambda i: (0, 0)),
            pl.BlockSpec((2 * H, H), lambda i: (0, 0)),
        ],
        out_specs=pl.BlockSpec((RB, H), lambda i: (i, 0)),
        out_shape=jax.ShapeDtypeStruct((N, H), jnp.float32),
    )(text_f, vis_f, W_t, b_t, W_v, b_v, W_g0)


def _tc_prescale(z, deg, N, n_pad):
    H = z.shape[1]

    def body(z_ref, dg_ref, o_ref, iv_ref):
        d = dg_ref[0] + dg_ref[1]
        inv = 1.0 / jnp.sqrt(jnp.maximum(d, 1.0))
        invt = jnp.transpose(inv, (1, 0))
        iv_ref[...] = invt
        o_ref[...] = z_ref[...] * invt[:N, 1:2]

    return pl.pallas_call(
        body,
        out_shape=(
            jax.ShapeDtypeStruct((N, H), jnp.float32),
            jax.ShapeDtypeStruct((n_pad, 2), jnp.float32),
        ),
    )(z, deg)


def _tc_mid(agg, invs, b_g0, W_g1, N):
    H = W_g1.shape[0]
    RB = 1000
    assert N % RB == 0

    def body(a_ref, iv_ref, b_ref, w_ref, o_ref):
        a = a_ref[:, :H] + a_ref[:, H:]
        iv = iv_ref[...]
        h = jnp.maximum(a * iv[:, 0:1] + b_ref[...], 0.0)
        z = jnp.dot(h, w_ref[...], preferred_element_type=jnp.float32)
        o_ref[...] = z * iv[:, 1:2]

    return pl.pallas_call(
        body,
        grid=(N // RB,),
        in_specs=[
            pl.BlockSpec((RB, NC * H), lambda i: (i, 0)),
            pl.BlockSpec((RB, 2), lambda i: (i, 0)),
            pl.BlockSpec((1, H), lambda i: (0, 0)),
            pl.BlockSpec((H, H), lambda i: (0, 0)),
        ],
        out_specs=pl.BlockSpec((RB, H), lambda i: (i, 0)),
        out_shape=jax.ShapeDtypeStruct((N, H), jnp.float32),
    )(agg, invs, b_g0, W_g1)


def _tc_head(agg, invs, b_g1, W_head, b_head, N):
    H, C = W_head.shape
    RB = 1000
    assert N % RB == 0

    def body(a_ref, iv_ref, b_ref, w_ref, bh_ref, o_ref):
        a = a_ref[:, :H] + a_ref[:, H:]
        iv = iv_ref[...]
        h = jnp.maximum(a * iv[:, 0:1] + b_ref[...], 0.0)
        o_ref[...] = (
            jnp.dot(h, w_ref[...], preferred_element_type=jnp.float32) + bh_ref[...]
        )

    return pl.pallas_call(
        body,
        grid=(N // RB,),
        in_specs=[
            pl.BlockSpec((RB, NC * H), lambda i: (i, 0)),
            pl.BlockSpec((RB, 2), lambda i: (i, 0)),
            pl.BlockSpec((1, H), lambda i: (0, 0)),
            pl.BlockSpec((H, C), lambda i: (0, 0)),
            pl.BlockSpec((1, C), lambda i: (0, 0)),
        ],
        out_specs=pl.BlockSpec((RB, C), lambda i: (i, 0)),
        out_shape=jax.ShapeDtypeStruct((N, C), jnp.float32),
    )(agg, invs, b_g1, W_head, b_head)


def kernel(edge_index, text_f, vis_f, W_t, b_t, W_v, b_v, W_g0, b_g0, W_g1, b_g1,
           W_head, b_head):
    N = text_f.shape[0]
    E = edge_index.shape[1]
    n_pad = -(-N // (NS * 64)) * (NS * 64)

    deg = _sc_degrees(edge_index, n_pad)
    z0 = _tc_encode(text_f, vis_f, W_t, b_t.reshape(1, -1), W_v,
                    b_v.reshape(1, -1), W_g0)
    p0, invs = _tc_prescale(z0, deg, N, n_pad)
    agg0 = _sc_aggregate(edge_index, p0, n_pad)
    p1 = _tc_mid(agg0, invs, b_g0.reshape(1, -1), W_g1, N)
    agg1 = _sc_aggregate(edge_index, p1, n_pad)
    out = _tc_head(agg1, invs, b_g1.reshape(1, -1), W_head, b_head.reshape(1, -1), N)
    return out


# speedup vs baseline: 1.1411x; 1.0907x over previous
import functools

import jax
import jax.numpy as jnp
from jax import lax
from jax.experimental import pallas as pl
from jax.experimental.pallas import tpu as pltpu
from jax.experimental.pallas import tpu_sc as plsc

NC = 2
NS = 16
NW = NC * NS
LANE = 128


def _mesh():
    return plsc.VectorSubcoreMesh(
        core_axis_name="c", subcore_axis_name="s", num_cores=NC, num_subcores=NS
    )


def _fill(ref, n, value):
    def body(i, _):
        ref[pl.ds(i * 16, 16)] = jnp.full((16,), value, jnp.float32)
        return 0
    lax.fori_loop(0, n // 16, body, 0)


def _sc_degrees(eflat, n_pad):
    E = eflat.shape[1]
    CH = 128
    assert E % NW == 0
    per_tile = E // NW
    full = per_tile // CH
    tail = per_tile % CH
    pairs, odd = divmod(full, 2)
    assert tail % 16 == 0 and per_tile % 8 == 0
    rpt = n_pad // NS

    def body(e_hbm, out_hbm, sbA, dbA, sbB, dbB, sbt, dbt, ones_v,
             onest_v, zer_v, din_sp, dout_sp, semA, semB, semC, semD,
             semS1, semS2, semS3, semS4):
        c = lax.axis_index("c")
        s = lax.axis_index("s")
        wid = c * NS + s
        base0 = wid * per_tile

        _fill(zer_v, rpt, 0.0)
        _fill(ones_v, CH, 1.0)
        if tail:
            _fill(onest_v, tail, 1.0)
        pltpu.sync_copy(zer_v, din_sp.at[pl.ds(s * rpt, rpt)])
        pltpu.sync_copy(zer_v, dout_sp.at[pl.ds(s * rpt, rpt)])
        plsc.subcore_barrier()

        def pbody(i, _):
            b0 = base0 + (2 * i) * CH
            b1 = b0 + CH
            dA = pltpu.async_copy(e_hbm.at[0, pl.ds(b0, CH)], sbA, semA)
            dB = pltpu.async_copy(e_hbm.at[1, pl.ds(b0, CH)], dbA, semB)
            dC = pltpu.async_copy(e_hbm.at[0, pl.ds(b1, CH)], sbB, semC)
            dD = pltpu.async_copy(e_hbm.at[1, pl.ds(b1, CH)], dbB, semD)
            dA.wait()
            s1 = pltpu.async_copy(ones_v, dout_sp.at[sbA], semS1, add=True)
            dB.wait()
            s2 = pltpu.async_copy(ones_v, din_sp.at[dbA], semS2, add=True)
            dC.wait()
            s3 = pltpu.async_copy(ones_v, dout_sp.at[sbB], semS3, add=True)
            dD.wait()
            s4 = pltpu.async_copy(ones_v, din_sp.at[dbB], semS4, add=True)
            s1.wait()
            s2.wait()
            s3.wait()
            s4.wait()
            return 0

        lax.fori_loop(0, pairs, pbody, 0)

        if odd:
            b0 = base0 + (pairs * 2) * CH
            pltpu.sync_copy(e_hbm.at[0, pl.ds(b0, CH)], sbA)
            pltpu.sync_copy(ones_v, dout_sp.at[sbA], add=True)
            pltpu.sync_copy(e_hbm.at[1, pl.ds(b0, CH)], dbA)
            pltpu.sync_copy(ones_v, din_sp.at[dbA], add=True)

        if tail:
            bt = base0 + full * CH
            pltpu.sync_copy(e_hbm.at[0, pl.ds(bt, tail)], sbt)
            pltpu.sync_copy(onest_v, dout_sp.at[sbt], add=True)
            pltpu.sync_copy(e_hbm.at[1, pl.ds(bt, tail)], dbt)
            pltpu.sync_copy(onest_v, din_sp.at[dbt], add=True)

        plsc.subcore_barrier()

        pltpu.sync_copy(din_sp.at[pl.ds(s * rpt, rpt)],
                        out_hbm.at[c, 0, pl.ds(s * rpt, rpt)])
        pltpu.sync_copy(dout_sp.at[pl.ds(s * rpt, rpt)],
                        out_hbm.at[c, 1, pl.ds(s * rpt, rpt)])

    f = pl.kernel(
        body,
        out_type=jax.ShapeDtypeStruct((NC, 2, n_pad), jnp.float32),
        mesh=_mesh(),
        scratch_types=[
            pltpu.VMEM((CH,), jnp.int32),
            pltpu.VMEM((CH,), jnp.int32),
            pltpu.VMEM((CH,), jnp.int32),
            pltpu.VMEM((CH,), jnp.int32),
            pltpu.VMEM((max(tail, 16),), jnp.int32),
            pltpu.VMEM((max(tail, 16),), jnp.int32),
            pltpu.VMEM((CH,), jnp.float32),
            pltpu.VMEM((max(tail, 16),), jnp.float32),
            pltpu.VMEM((rpt,), jnp.float32),
            pltpu.VMEM_SHARED((n_pad,), jnp.float32),
            pltpu.VMEM_SHARED((n_pad,), jnp.float32),
            pltpu.SemaphoreType.DMA,
            pltpu.SemaphoreType.DMA,
            pltpu.SemaphoreType.DMA,
            pltpu.SemaphoreType.DMA,
            pltpu.SemaphoreType.DMA,
            pltpu.SemaphoreType.DMA,
            pltpu.SemaphoreType.DMA,
            pltpu.SemaphoreType.DMA,
        ],
        compiler_params=pltpu.CompilerParams(use_tc_tiling_on_sc=False),
    )
    return f(eflat)


def _sc_aggregate(eflat, p, n_pad):
    E = eflat.shape[1]
    H = p.shape[1]
    CH = 128
    NB = 6
    assert E % NW == 0
    per_tile = E // NW
    full = per_tile // CH
    tail = per_tile % CH
    rounds, rem = divmod(full, NB)
    assert tail % 8 == 0 and per_tile % 8 == 0
    rpt = n_pad // NS
    ZR = 64
    assert rpt % ZR == 0

    def body(e_hbm, p_hbm, out_hbm, sbig, dbt, rowst, zer_v, acc_sp, *rest):
        dbs = rest[0:NB]
        rows_l = rest[NB:2 * NB]
        semD = rest[2 * NB:3 * NB]
        semG = rest[3 * NB:4 * NB]
        semS = rest[4 * NB:5 * NB]
        c = lax.axis_index("c")
        s = lax.axis_index("s")
        wid = c * NS + s
        base0 = wid * per_tile

        pre = pltpu.async_copy(e_hbm.at[0, pl.ds(base0, per_tile)], sbig, semG[0])

        def zfill(i, _):
            zer_v[i, pl.ds(0, 16)] = jnp.zeros((16,), jnp.float32)
            zer_v[i, pl.ds(16, 16)] = jnp.zeros((16,), jnp.float32)
            zer_v[i, pl.ds(32, 16)] = jnp.zeros((16,), jnp.float32)
            zer_v[i, pl.ds(48, 16)] = jnp.zeros((16,), jnp.float32)
            return 0
        lax.fori_loop(0, ZR, zfill, 0)

        def zcopy(i, _):
            pltpu.sync_copy(zer_v, acc_sp.at[pl.ds(s * rpt + i * ZR, ZR)])
            return 0
        lax.fori_loop(0, rpt // ZR, zcopy, 0)
        pre.wait()
        plsc.subcore_barrier()

        bufs = tuple(zip(dbs, rows_l, semD, semG, semS))

        def tbody(i, _):
            descs = []
            for k, (db, rows, sD, sG, sS) in enumerate(bufs):
                b0 = (NB * i + k) * CH
                @pl.when(i > 0)
                def _(db=db, rows=rows, sS=sS):
                    pltpu.make_async_copy(rows, acc_sp.at[db], sS).wait()
                descs.append((
                    pltpu.async_copy(e_hbm.at[1, pl.ds(base0 + b0, CH)], db, sD),
                    pltpu.async_copy(p_hbm.at[sbig.at[pl.ds(b0, CH)]], rows, sG),
                ))
            for (d, g), (db, rows, _, _, sS) in zip(descs, bufs):
                d.wait()
                g.wait()
                pltpu.async_copy(rows, acc_sp.at[db], sS, add=True)
            return 0

        lax.fori_loop(0, rounds, tbody, 0)
        if rounds > 0:
            for db, rows, _, _, sS in bufs:
                pltpu.make_async_copy(rows, acc_sp.at[db], sS).wait()

        for r in range(rem):
            b0 = (rounds * NB + r) * CH
            db, rows, sD, sG, sS = bufs[0]
            pltpu.sync_copy(e_hbm.at[1, pl.ds(base0 + b0, CH)], db)
            pltpu.async_copy(p_hbm.at[sbig.at[pl.ds(b0, CH)]], rows, sG).wait()
            pltpu.sync_copy(rows, acc_sp.at[db], add=True)

        if tail:
            bt = full * CH
            pltpu.sync_copy(e_hbm.at[1, pl.ds(base0 + bt, tail)], dbt)
            pltpu.async_copy(p_hbm.at[sbig.at[pl.ds(bt, tail)]], rowst,
                             semG[0]).wait()
            pltpu.sync_copy(rowst, acc_sp.at[dbt], add=True)

        plsc.subcore_barrier()
        pltpu.sync_copy(acc_sp.at[pl.ds(s * rpt, rpt)],
                        out_hbm.at[pl.ds(s * rpt, rpt), pl.ds(c * H, H)])

    f = pl.kernel(
        body,
        out_type=jax.ShapeDtypeStruct((n_pad, NC * H), jnp.float32),
        mesh=_mesh(),
        scratch_types=(
            [
                pltpu.VMEM((per_tile,), jnp.int32),
                pltpu.VMEM((max(tail, 8),), jnp.int32),
                pltpu.VMEM((max(tail, 8), H), jnp.float32),
                pltpu.VMEM((ZR, H), jnp.float32),
                pltpu.VMEM_SHARED((n_pad, H), jnp.float32),
            ]
            + [pltpu.VMEM((CH,), jnp.int32)] * NB
            + [pltpu.VMEM((CH, H), jnp.float32)] * NB
            + [pltpu.SemaphoreType.DMA] * (3 * NB)
        ),
        compiler_params=pltpu.CompilerParams(use_tc_tiling_on_sc=False),
    )
    return f(eflat, p)


def _tc_encode(text_f, vis_f, W_t, b_t, W_v, b_v, W_g0):
    N, T = text_f.shape
    V = vis_f.shape[1]
    H = W_t.shape[1]
    RB = 1000
    assert N % RB == 0

    def body(t_ref, v_ref, wt_ref, bt_ref, wv_ref, bv_ref, wg_ref, o_ref):
        ht = jnp.maximum(
            jnp.dot(t_ref[...], wt_ref[...], preferred_element_type=jnp.float32)
            + bt_ref[...], 0.0)
        hv = jnp.maximum(
            jnp.dot(v_ref[...], wv_ref[...], preferred_element_type=jnp.float32)
            + bv_ref[...], 0.0)
        h = jnp.concatenate([ht, hv], axis=1)
        o_ref[...] = jnp.dot(h, wg_ref[...], preferred_element_type=jnp.float32)

    return pl.pallas_call(
        body,
        grid=(N // RB,),
        in_specs=[
            pl.BlockSpec((RB, T), lambda i: (i, 0)),
            pl.BlockSpec((RB, V), lambda i: (i, 0)),
            pl.BlockSpec((T, H), lambda i: (0, 0)),
            pl.BlockSpec((1, H), lambda i: (0, 0)),
            pl.BlockSpec((V, H), lambda i: (0, 0)),
            pl.BlockSpec((1, H), lambda i: (0, 0)),
            pl.BlockSpec((2 * H, H), lambda i: (0, 0)),
        ],
        out_specs=pl.BlockSpec((RB, H), lambda i: (i, 0)),
        out_shape=jax.ShapeDtypeStruct((N, H), jnp.float32),
    )(text_f, vis_f, W_t, b_t, W_v, b_v, W_g0)


def _tc_prescale(z, deg, N, n_pad):
    H = z.shape[1]

    def body(z_ref, dg_ref, o_ref, iv_ref):
        d = dg_ref[0] + dg_ref[1]
        inv = 1.0 / jnp.sqrt(jnp.maximum(d, 1.0))
        invt = jnp.transpose(inv, (1, 0))
        iv_ref[...] = invt
        o_ref[...] = z_ref[...] * invt[:N, 1:2]

    return pl.pallas_call(
        body,
        out_shape=(
            jax.ShapeDtypeStruct((N, H), jnp.float32),
            jax.ShapeDtypeStruct((n_pad, 2), jnp.float32),
        ),
    )(z, deg)


def _tc_mid(agg, invs, b_g0, W_g1, N):
    H = W_g1.shape[0]
    RB = 1000
    assert N % RB == 0

    def body(a_ref, iv_ref, b_ref, w_ref, o_ref):
        a = a_ref[:, :H] + a_ref[:, H:]
        iv = iv_ref[...]
        h = jnp.maximum(a * iv[:, 0:1] + b_ref[...], 0.0)
        z = jnp.dot(h, w_ref[...], preferred_element_type=jnp.float32)
        o_ref[...] = z * iv[:, 1:2]

    return pl.pallas_call(
        body,
        grid=(N // RB,),
        in_specs=[
            pl.BlockSpec((RB, NC * H), lambda i: (i, 0)),
            pl.BlockSpec((RB, 2), lambda i: (i, 0)),
            pl.BlockSpec((1, H), lambda i: (0, 0)),
            pl.BlockSpec((H, H), lambda i: (0, 0)),
        ],
        out_specs=pl.BlockSpec((RB, H), lambda i: (i, 0)),
        out_shape=jax.ShapeDtypeStruct((N, H), jnp.float32),
    )(agg, invs, b_g0, W_g1)


def _tc_head(agg, invs, b_g1, W_head, b_head, N):
    H, C = W_head.shape
    RB = 1000
    assert N % RB == 0

    def body(a_ref, iv_ref, b_ref, w_ref, bh_ref, o_ref):
        a = a_ref[:, :H] + a_ref[:, H:]
        iv = iv_ref[...]
        h = jnp.maximum(a * iv[:, 0:1] + b_ref[...], 0.0)
        o_ref[...] = (
            jnp.dot(h, w_ref[...], preferred_element_type=jnp.float32) + bh_ref[...]
        )

    return pl.pallas_call(
        body,
        grid=(N // RB,),
        in_specs=[
            pl.BlockSpec((RB, NC * H), lambda i: (i, 0)),
            pl.BlockSpec((RB, 2), lambda i: (i, 0)),
            pl.BlockSpec((1, H), lambda i: (0, 0)),
            pl.BlockSpec((H, C), lambda i: (0, 0)),
            pl.BlockSpec((1, C), lambda i: (0, 0)),
        ],
        out_specs=pl.BlockSpec((RB, C), lambda i: (i, 0)),
        out_shape=jax.ShapeDtypeStruct((N, C), jnp.float32),
    )(agg, invs, b_g1, W_head, b_head)


def kernel(edge_index, text_f, vis_f, W_t, b_t, W_v, b_v, W_g0, b_g0, W_g1, b_g1,
           W_head, b_head):
    N = text_f.shape[0]
    E = edge_index.shape[1]
    n_pad = -(-N // (NS * 64)) * (NS * 64)

    deg = _sc_degrees(edge_index, n_pad)
    z0 = _tc_encode(text_f, vis_f, W_t, b_t.reshape(1, -1), W_v,
                    b_v.reshape(1, -1), W_g0)
    p0, invs = _tc_prescale(z0, deg, N, n_pad)
    agg0 = _sc_aggregate(edge_index, p0, n_pad)
    p1 = _tc_mid(agg0, invs, b_g0.reshape(1, -1), W_g1, N)
    agg1 = _sc_aggregate(edge_index, p1, n_pad)
    out = _tc_head(agg1, invs, b_g1.reshape(1, -1), W_head, b_head.reshape(1, -1), N)
    return out


# speedup vs baseline: 1.2271x; 1.0754x over previous
import functools

import jax
import jax.numpy as jnp
from jax import lax
from jax.experimental import pallas as pl
from jax.experimental.pallas import tpu as pltpu
from jax.experimental.pallas import tpu_sc as plsc

NC = 2
NS = 16
NW = NC * NS
LANE = 128


def _mesh():
    return plsc.VectorSubcoreMesh(
        core_axis_name="c", subcore_axis_name="s", num_cores=NC, num_subcores=NS
    )


def _fill(ref, n, value):
    def body(i, _):
        ref[pl.ds(i * 16, 16)] = jnp.full((16,), value, jnp.float32)
        return 0
    lax.fori_loop(0, n // 16, body, 0)


def _sc_degrees(eflat, n_pad):
    E = eflat.shape[1]
    CH = 128
    NB = 6
    assert E % NW == 0
    per_tile = E // NW
    full = per_tile // CH
    tail = per_tile % CH
    rounds, rem = divmod(full, NB)
    assert tail % 16 == 0 and per_tile % 8 == 0
    rpt = n_pad // NS

    def body(e_hbm, out_hbm, sbt, dbt, ones_v, onest_v, zer_v,
             din_sp, dout_sp, *rest):
        sbs = rest[0:NB]
        dbs = rest[NB:2 * NB]
        semA = rest[2 * NB:3 * NB]
        semB = rest[3 * NB:4 * NB]
        semS1 = rest[4 * NB:5 * NB]
        semS2 = rest[5 * NB:6 * NB]
        c = lax.axis_index("c")
        s = lax.axis_index("s")
        wid = c * NS + s
        base0 = wid * per_tile

        _fill(zer_v, rpt, 0.0)
        _fill(ones_v, CH, 1.0)
        if tail:
            _fill(onest_v, tail, 1.0)
        pltpu.sync_copy(zer_v, din_sp.at[pl.ds(s * rpt, rpt)])
        pltpu.sync_copy(zer_v, dout_sp.at[pl.ds(s * rpt, rpt)])
        plsc.subcore_barrier()

        bufs = tuple(zip(sbs, dbs, semA, semB, semS1, semS2))

        def pbody(i, _):
            descs = []
            for k, (sb, db, sA, sB, sS1, sS2) in enumerate(bufs):
                b0 = base0 + (NB * i + k) * CH

                @pl.when(i > 0)
                def _(sb=sb, db=db, sS1=sS1, sS2=sS2):
                    pltpu.make_async_copy(ones_v, dout_sp.at[sb], sS1).wait()
                    pltpu.make_async_copy(ones_v, din_sp.at[db], sS2).wait()
                descs.append((
                    pltpu.async_copy(e_hbm.at[0, pl.ds(b0, CH)], sb, sA),
                    pltpu.async_copy(e_hbm.at[1, pl.ds(b0, CH)], db, sB),
                ))
            for (dA, dB), (sb, db, _, _, sS1, sS2) in zip(descs, bufs):
                dA.wait()
                pltpu.async_copy(ones_v, dout_sp.at[sb], sS1, add=True)
                dB.wait()
                pltpu.async_copy(ones_v, din_sp.at[db], sS2, add=True)
            return 0

        lax.fori_loop(0, rounds, pbody, 0)
        if rounds > 0:
            for sb, db, _, _, sS1, sS2 in bufs:
                pltpu.make_async_copy(ones_v, dout_sp.at[sb], sS1).wait()
                pltpu.make_async_copy(ones_v, din_sp.at[db], sS2).wait()

        for r in range(rem):
            b0 = base0 + (rounds * NB + r) * CH
            sb, db = bufs[0][0], bufs[0][1]
            pltpu.sync_copy(e_hbm.at[0, pl.ds(b0, CH)], sb)
            pltpu.sync_copy(ones_v, dout_sp.at[sb], add=True)
            pltpu.sync_copy(e_hbm.at[1, pl.ds(b0, CH)], db)
            pltpu.sync_copy(ones_v, din_sp.at[db], add=True)

        if tail:
            bt = base0 + full * CH
            pltpu.sync_copy(e_hbm.at[0, pl.ds(bt, tail)], sbt)
            pltpu.sync_copy(onest_v, dout_sp.at[sbt], add=True)
            pltpu.sync_copy(e_hbm.at[1, pl.ds(bt, tail)], dbt)
            pltpu.sync_copy(onest_v, din_sp.at[dbt], add=True)

        plsc.subcore_barrier()

        pltpu.sync_copy(din_sp.at[pl.ds(s * rpt, rpt)],
                        out_hbm.at[c, 0, pl.ds(s * rpt, rpt)])
        pltpu.sync_copy(dout_sp.at[pl.ds(s * rpt, rpt)],
                        out_hbm.at[c, 1, pl.ds(s * rpt, rpt)])

    f = pl.kernel(
        body,
        out_type=jax.ShapeDtypeStruct((NC, 2, n_pad), jnp.float32),
        mesh=_mesh(),
        scratch_types=(
            [
                pltpu.VMEM((max(tail, 16),), jnp.int32),
                pltpu.VMEM((max(tail, 16),), jnp.int32),
                pltpu.VMEM((CH,), jnp.float32),
                pltpu.VMEM((max(tail, 16),), jnp.float32),
                pltpu.VMEM((rpt,), jnp.float32),
                pltpu.VMEM_SHARED((n_pad,), jnp.float32),
                pltpu.VMEM_SHARED((n_pad,), jnp.float32),
            ]
            + [pltpu.VMEM((CH,), jnp.int32)] * (2 * NB)
            + [pltpu.SemaphoreType.DMA] * (4 * NB)
        ),
        compiler_params=pltpu.CompilerParams(use_tc_tiling_on_sc=False),
    )
    return f(eflat)


def _sc_aggregate(eflat, p, n_pad):
    E = eflat.shape[1]
    H = p.shape[1]
    CH = 128
    NB = 6
    assert E % NW == 0
    per_tile = E // NW
    full = per_tile // CH
    tail = per_tile % CH
    rounds, rem = divmod(full, NB)
    assert tail % 8 == 0 and per_tile % 8 == 0
    rpt = n_pad // NS
    ZR = 64
    assert rpt % ZR == 0

    def body(e_hbm, p_hbm, out_hbm, sbig, dbt, rowst, zer_v, acc_sp, *rest):
        dbs = rest[0:NB]
        rows_l = rest[NB:2 * NB]
        semD = rest[2 * NB:3 * NB]
        semG = rest[3 * NB:4 * NB]
        semS = rest[4 * NB:5 * NB]
        c = lax.axis_index("c")
        s = lax.axis_index("s")
        wid = c * NS + s
        base0 = wid * per_tile

        pre = pltpu.async_copy(e_hbm.at[0, pl.ds(base0, per_tile)], sbig, semG[0])

        def zfill(i, _):
            zer_v[i, pl.ds(0, 16)] = jnp.zeros((16,), jnp.float32)
            zer_v[i, pl.ds(16, 16)] = jnp.zeros((16,), jnp.float32)
            zer_v[i, pl.ds(32, 16)] = jnp.zeros((16,), jnp.float32)
            zer_v[i, pl.ds(48, 16)] = jnp.zeros((16,), jnp.float32)
            return 0
        lax.fori_loop(0, ZR, zfill, 0)

        def zcopy(i, _):
            pltpu.sync_copy(zer_v, acc_sp.at[pl.ds(s * rpt + i * ZR, ZR)])
            return 0
        lax.fori_loop(0, rpt // ZR, zcopy, 0)
        pre.wait()
        plsc.subcore_barrier()

        bufs = tuple(zip(dbs, rows_l, semD, semG, semS))

        def tbody(i, _):
            descs = []
            for k, (db, rows, sD, sG, sS) in enumerate(bufs):
                b0 = (NB * i + k) * CH
                @pl.when(i > 0)
                def _(db=db, rows=rows, sS=sS):
                    pltpu.make_async_copy(rows, acc_sp.at[db], sS).wait()
                descs.append((
                    pltpu.async_copy(e_hbm.at[1, pl.ds(base0 + b0, CH)], db, sD),
                    pltpu.async_copy(p_hbm.at[sbig.at[pl.ds(b0, CH)]], rows, sG),
                ))
            for (d, g), (db, rows, _, _, sS) in zip(descs, bufs):
                d.wait()
                g.wait()
                pltpu.async_copy(rows, acc_sp.at[db], sS, add=True)
            return 0

        lax.fori_loop(0, rounds, tbody, 0)
        if rounds > 0:
            for db, rows, _, _, sS in bufs:
                pltpu.make_async_copy(rows, acc_sp.at[db], sS).wait()

        for r in range(rem):
            b0 = (rounds * NB + r) * CH
            db, rows, sD, sG, sS = bufs[0]
            pltpu.sync_copy(e_hbm.at[1, pl.ds(base0 + b0, CH)], db)
            pltpu.async_copy(p_hbm.at[sbig.at[pl.ds(b0, CH)]], rows, sG).wait()
            pltpu.sync_copy(rows, acc_sp.at[db], add=True)

        if tail:
            bt = full * CH
            pltpu.sync_copy(e_hbm.at[1, pl.ds(base0 + bt, tail)], dbt)
            pltpu.async_copy(p_hbm.at[sbig.at[pl.ds(bt, tail)]], rowst,
                             semG[0]).wait()
            pltpu.sync_copy(rowst, acc_sp.at[dbt], add=True)

        plsc.subcore_barrier()
        pltpu.sync_copy(acc_sp.at[pl.ds(s * rpt, rpt)],
                        out_hbm.at[pl.ds(s * rpt, rpt), pl.ds(c * H, H)])

    f = pl.kernel(
        body,
        out_type=jax.ShapeDtypeStruct((n_pad, NC * H), jnp.float32),
        mesh=_mesh(),
        scratch_types=(
            [
                pltpu.VMEM((per_tile,), jnp.int32),
                pltpu.VMEM((max(tail, 8),), jnp.int32),
                pltpu.VMEM((max(tail, 8), H), jnp.float32),
                pltpu.VMEM((ZR, H), jnp.float32),
                pltpu.VMEM_SHARED((n_pad, H), jnp.float32),
            ]
            + [pltpu.VMEM((CH,), jnp.int32)] * NB
            + [pltpu.VMEM((CH, H), jnp.float32)] * NB
            + [pltpu.SemaphoreType.DMA] * (3 * NB)
        ),
        compiler_params=pltpu.CompilerParams(use_tc_tiling_on_sc=False),
    )
    return f(eflat, p)


def _tc_encode(text_f, vis_f, W_t, b_t, W_v, b_v, W_g0):
    N, T = text_f.shape
    V = vis_f.shape[1]
    H = W_t.shape[1]
    RB = 1000
    assert N % RB == 0

    def body(t_ref, v_ref, wt_ref, bt_ref, wv_ref, bv_ref, wg_ref, o_ref):
        ht = jnp.maximum(
            jnp.dot(t_ref[...], wt_ref[...], preferred_element_type=jnp.float32)
            + bt_ref[...], 0.0)
        hv = jnp.maximum(
            jnp.dot(v_ref[...], wv_ref[...], preferred_element_type=jnp.float32)
            + bv_ref[...], 0.0)
        h = jnp.concatenate([ht, hv], axis=1)
        o_ref[...] = jnp.dot(h, wg_ref[...], preferred_element_type=jnp.float32)

    return pl.pallas_call(
        body,
        grid=(N // RB,),
        in_specs=[
            pl.BlockSpec((RB, T), lambda i: (i, 0)),
            pl.BlockSpec((RB, V), lambda i: (i, 0)),
            pl.BlockSpec((T, H), lambda i: (0, 0)),
            pl.BlockSpec((1, H), lambda i: (0, 0)),
            pl.BlockSpec((V, H), lambda i: (0, 0)),
            pl.BlockSpec((1, H), lambda i: (0, 0)),
            pl.BlockSpec((2 * H, H), lambda i: (0, 0)),
        ],
        out_specs=pl.BlockSpec((RB, H), lambda i: (i, 0)),
        out_shape=jax.ShapeDtypeStruct((N, H), jnp.float32),
    )(text_f, vis_f, W_t, b_t, W_v, b_v, W_g0)


def _tc_prescale(z, deg, N, n_pad):
    H = z.shape[1]

    def body(z_ref, dg_ref, o_ref, iv_ref):
        d = dg_ref[0] + dg_ref[1]
        inv = 1.0 / jnp.sqrt(jnp.maximum(d, 1.0))
        invt = jnp.transpose(inv, (1, 0))
        iv_ref[...] = invt
        o_ref[...] = z_ref[...] * invt[:N, 1:2]

    return pl.pallas_call(
        body,
        out_shape=(
            jax.ShapeDtypeStruct((N, H), jnp.float32),
            jax.ShapeDtypeStruct((n_pad, 2), jnp.float32),
        ),
    )(z, deg)


def _tc_mid(agg, invs, b_g0, W_g1, N):
    H = W_g1.shape[0]
    RB = 1000
    assert N % RB == 0

    def body(a_ref, iv_ref, b_ref, w_ref, o_ref):
        a = a_ref[:, :H] + a_ref[:, H:]
        iv = iv_ref[...]
        h = jnp.maximum(a * iv[:, 0:1] + b_ref[...], 0.0)
        z = jnp.dot(h, w_ref[...], preferred_element_type=jnp.float32)
        o_ref[...] = z * iv[:, 1:2]

    return pl.pallas_call(
        body,
        grid=(N // RB,),
        in_specs=[
            pl.BlockSpec((RB, NC * H), lambda i: (i, 0)),
            pl.BlockSpec((RB, 2), lambda i: (i, 0)),
            pl.BlockSpec((1, H), lambda i: (0, 0)),
            pl.BlockSpec((H, H), lambda i: (0, 0)),
        ],
        out_specs=pl.BlockSpec((RB, H), lambda i: (i, 0)),
        out_shape=jax.ShapeDtypeStruct((N, H), jnp.float32),
    )(agg, invs, b_g0, W_g1)


def _tc_head(agg, invs, b_g1, W_head, b_head, N):
    H, C = W_head.shape
    RB = 1000
    assert N % RB == 0

    def body(a_ref, iv_ref, b_ref, w_ref, bh_ref, o_ref):
        a = a_ref[:, :H] + a_ref[:, H:]
        iv = iv_ref[...]
        h = jnp.maximum(a * iv[:, 0:1] + b_ref[...], 0.0)
        o_ref[...] = (
            jnp.dot(h, w_ref[...], preferred_element_type=jnp.float32) + bh_ref[...]
        )

    return pl.pallas_call(
        body,
        grid=(N // RB,),
        in_specs=[
            pl.BlockSpec((RB, NC * H), lambda i: (i, 0)),
            pl.BlockSpec((RB, 2), lambda i: (i, 0)),
            pl.BlockSpec((1, H), lambda i: (0, 0)),
            pl.BlockSpec((H, C), lambda i: (0, 0)),
            pl.BlockSpec((1, C), lambda i: (0, 0)),
        ],
        out_specs=pl.BlockSpec((RB, C), lambda i: (i, 0)),
        out_shape=jax.ShapeDtypeStruct((N, C), jnp.float32),
    )(agg, invs, b_g1, W_head, b_head)


def kernel(edge_index, text_f, vis_f, W_t, b_t, W_v, b_v, W_g0, b_g0, W_g1, b_g1,
           W_head, b_head):
    N = text_f.shape[0]
    E = edge_index.shape[1]
    n_pad = -(-N // (NS * 64)) * (NS * 64)

    deg = _sc_degrees(edge_index, n_pad)
    z0 = _tc_encode(text_f, vis_f, W_t, b_t.reshape(1, -1), W_v,
                    b_v.reshape(1, -1), W_g0)
    p0, invs = _tc_prescale(z0, deg, N, n_pad)
    agg0 = _sc_aggregate(edge_index, p0, n_pad)
    p1 = _tc_mid(agg0, invs, b_g0.reshape(1, -1), W_g1, N)
    agg1 = _sc_aggregate(edge_index, p1, n_pad)
    out = _tc_head(agg1, invs, b_g1.reshape(1, -1), W_head, b_head.reshape(1, -1), N)
    return out


# speedup vs baseline: 1.2513x; 1.0197x over previous
import functools

import jax
import jax.numpy as jnp
from jax import lax
from jax.experimental import pallas as pl
from jax.experimental.pallas import tpu as pltpu
from jax.experimental.pallas import tpu_sc as plsc

NC = 2
NS = 16
NW = NC * NS
LANE = 128


def _mesh():
    return plsc.VectorSubcoreMesh(
        core_axis_name="c", subcore_axis_name="s", num_cores=NC, num_subcores=NS
    )


def _fill(ref, n, value):
    def body(i, _):
        ref[pl.ds(i * 16, 16)] = jnp.full((16,), value, jnp.float32)
        return 0
    lax.fori_loop(0, n // 16, body, 0)


def _sc_degrees(eflat, n_pad):
    E = eflat.shape[1]
    CH = 128
    NB = 6
    assert E % NW == 0
    per_tile = E // NW
    full = per_tile // CH
    tail = per_tile % CH
    rounds, rem = divmod(full, NB)
    assert tail % 16 == 0 and per_tile % 8 == 0
    rpt = n_pad // NS

    def body(e_hbm, out_hbm, sbt, dbt, ones_v, onest_v, zer_v,
             din_sp, dout_sp, *rest):
        sbs = rest[0:NB]
        dbs = rest[NB:2 * NB]
        semA = rest[2 * NB:3 * NB]
        semB = rest[3 * NB:4 * NB]
        semS1 = rest[4 * NB:5 * NB]
        semS2 = rest[5 * NB:6 * NB]
        c = lax.axis_index("c")
        s = lax.axis_index("s")
        wid = c * NS + s
        base0 = wid * per_tile

        _fill(zer_v, rpt, 0.0)
        _fill(ones_v, CH, 1.0)
        if tail:
            _fill(onest_v, tail, 1.0)
        pltpu.sync_copy(zer_v, din_sp.at[pl.ds(s * rpt, rpt)])
        pltpu.sync_copy(zer_v, dout_sp.at[pl.ds(s * rpt, rpt)])
        plsc.subcore_barrier()

        bufs = tuple(zip(sbs, dbs, semA, semB, semS1, semS2))

        def pbody(i, _):
            descs = []
            for k, (sb, db, sA, sB, sS1, sS2) in enumerate(bufs):
                b0 = base0 + (NB * i + k) * CH

                @pl.when(i > 0)
                def _(sb=sb, db=db, sS1=sS1, sS2=sS2):
                    pltpu.make_async_copy(ones_v, dout_sp.at[sb], sS1).wait()
                    pltpu.make_async_copy(ones_v, din_sp.at[db], sS2).wait()
                descs.append((
                    pltpu.async_copy(e_hbm.at[0, pl.ds(b0, CH)], sb, sA),
                    pltpu.async_copy(e_hbm.at[1, pl.ds(b0, CH)], db, sB),
                ))
            for (dA, dB), (sb, db, _, _, sS1, sS2) in zip(descs, bufs):
                dA.wait()
                pltpu.async_copy(ones_v, dout_sp.at[sb], sS1, add=True)
                dB.wait()
                pltpu.async_copy(ones_v, din_sp.at[db], sS2, add=True)
            return 0

        lax.fori_loop(0, rounds, pbody, 0)
        if rounds > 0:
            for sb, db, _, _, sS1, sS2 in bufs:
                pltpu.make_async_copy(ones_v, dout_sp.at[sb], sS1).wait()
                pltpu.make_async_copy(ones_v, din_sp.at[db], sS2).wait()

        for r in range(rem):
            b0 = base0 + (rounds * NB + r) * CH
            sb, db = bufs[0][0], bufs[0][1]
            pltpu.sync_copy(e_hbm.at[0, pl.ds(b0, CH)], sb)
            pltpu.sync_copy(ones_v, dout_sp.at[sb], add=True)
            pltpu.sync_copy(e_hbm.at[1, pl.ds(b0, CH)], db)
            pltpu.sync_copy(ones_v, din_sp.at[db], add=True)

        if tail:
            bt = base0 + full * CH
            pltpu.sync_copy(e_hbm.at[0, pl.ds(bt, tail)], sbt)
            pltpu.sync_copy(onest_v, dout_sp.at[sbt], add=True)
            pltpu.sync_copy(e_hbm.at[1, pl.ds(bt, tail)], dbt)
            pltpu.sync_copy(onest_v, din_sp.at[dbt], add=True)

        plsc.subcore_barrier()

        pltpu.sync_copy(din_sp.at[pl.ds(s * rpt, rpt)],
                        out_hbm.at[c, 0, pl.ds(s * rpt, rpt)])
        pltpu.sync_copy(dout_sp.at[pl.ds(s * rpt, rpt)],
                        out_hbm.at[c, 1, pl.ds(s * rpt, rpt)])

    f = pl.kernel(
        body,
        out_type=jax.ShapeDtypeStruct((NC, 2, n_pad), jnp.float32),
        mesh=_mesh(),
        scratch_types=(
            [
                pltpu.VMEM((max(tail, 16),), jnp.int32),
                pltpu.VMEM((max(tail, 16),), jnp.int32),
                pltpu.VMEM((CH,), jnp.float32),
                pltpu.VMEM((max(tail, 16),), jnp.float32),
                pltpu.VMEM((rpt,), jnp.float32),
                pltpu.VMEM_SHARED((n_pad,), jnp.float32),
                pltpu.VMEM_SHARED((n_pad,), jnp.float32),
            ]
            + [pltpu.VMEM((CH,), jnp.int32)] * (2 * NB)
            + [pltpu.SemaphoreType.DMA] * (4 * NB)
        ),
        compiler_params=pltpu.CompilerParams(use_tc_tiling_on_sc=False),
    )
    return f(eflat)


def _sc_aggregate(eflat, p, n_pad):
    E = eflat.shape[1]
    H = p.shape[1]
    CH = 128
    NB = 6
    assert E % NW == 0
    per_tile = E // NW
    full = per_tile // CH
    tail = per_tile % CH
    rounds, rem = divmod(full, NB)
    assert tail % 8 == 0 and per_tile % 8 == 0
    rpt = n_pad // NS
    ZR = 64
    assert rpt % ZR == 0

    def body(e_hbm, p_hbm, out_hbm, sbig, dbt, rowst, zer_v, acc_sp, *rest):
        dbs = rest[0:NB]
        rows_l = rest[NB:2 * NB]
        semD = rest[2 * NB:3 * NB]
        semG = rest[3 * NB:4 * NB]
        semS = rest[4 * NB:5 * NB]
        c = lax.axis_index("c")
        s = lax.axis_index("s")
        wid = c * NS + s
        base0 = wid * per_tile

        pre = pltpu.async_copy(e_hbm.at[0, pl.ds(base0, per_tile)], sbig, semG[0])

        def zfill(i, _):
            zer_v[i, pl.ds(0, 16)] = jnp.zeros((16,), jnp.float32)
            zer_v[i, pl.ds(16, 16)] = jnp.zeros((16,), jnp.float32)
            zer_v[i, pl.ds(32, 16)] = jnp.zeros((16,), jnp.float32)
            zer_v[i, pl.ds(48, 16)] = jnp.zeros((16,), jnp.float32)
            return 0
        lax.fori_loop(0, ZR, zfill, 0)

        def zcopy(i, _):
            pltpu.sync_copy(zer_v, acc_sp.at[pl.ds(s * rpt + i * ZR, ZR)])
            return 0
        lax.fori_loop(0, rpt // ZR, zcopy, 0)
        pre.wait()
        plsc.subcore_barrier()

        bufs = tuple(zip(dbs, rows_l, semD, semG, semS))

        def tbody(i, _):
            descs = []
            for k, (db, rows, sD, sG, sS) in enumerate(bufs):
                b0 = (NB * i + k) * CH
                @pl.when(i > 0)
                def _(db=db, rows=rows, sS=sS):
                    pltpu.make_async_copy(rows, acc_sp.at[db], sS).wait()
                descs.append((
                    pltpu.async_copy(e_hbm.at[1, pl.ds(base0 + b0, CH)], db, sD),
                    pltpu.async_copy(p_hbm.at[sbig.at[pl.ds(b0, CH)]], rows, sG),
                ))
            for (d, g), (db, rows, _, _, sS) in zip(descs, bufs):
                d.wait()
                g.wait()
                pltpu.async_copy(rows, acc_sp.at[db], sS, add=True)
            return 0

        lax.fori_loop(0, rounds, tbody, 0)
        if rounds > 0:
            for db, rows, _, _, sS in bufs:
                pltpu.make_async_copy(rows, acc_sp.at[db], sS).wait()

        for r in range(rem):
            b0 = (rounds * NB + r) * CH
            db, rows, sD, sG, sS = bufs[0]
            pltpu.sync_copy(e_hbm.at[1, pl.ds(base0 + b0, CH)], db)
            pltpu.async_copy(p_hbm.at[sbig.at[pl.ds(b0, CH)]], rows, sG).wait()
            pltpu.sync_copy(rows, acc_sp.at[db], add=True)

        if tail:
            bt = full * CH
            pltpu.sync_copy(e_hbm.at[1, pl.ds(base0 + bt, tail)], dbt)
            pltpu.async_copy(p_hbm.at[sbig.at[pl.ds(bt, tail)]], rowst,
                             semG[0]).wait()
            pltpu.sync_copy(rowst, acc_sp.at[dbt], add=True)

        plsc.subcore_barrier()
        pltpu.sync_copy(acc_sp.at[pl.ds(s * rpt, rpt)],
                        out_hbm.at[pl.ds(s * rpt, rpt), pl.ds(c * H, H)])

    f = pl.kernel(
        body,
        out_type=jax.ShapeDtypeStruct((n_pad, NC * H), jnp.float32),
        mesh=_mesh(),
        scratch_types=(
            [
                pltpu.VMEM((per_tile,), jnp.int32),
                pltpu.VMEM((max(tail, 8),), jnp.int32),
                pltpu.VMEM((max(tail, 8), H), jnp.float32),
                pltpu.VMEM((ZR, H), jnp.float32),
                pltpu.VMEM_SHARED((n_pad, H), jnp.float32),
            ]
            + [pltpu.VMEM((CH,), jnp.int32)] * NB
            + [pltpu.VMEM((CH, H), jnp.float32)] * NB
            + [pltpu.SemaphoreType.DMA] * (3 * NB)
        ),
        compiler_params=pltpu.CompilerParams(use_tc_tiling_on_sc=False),
    )
    return f(eflat, p)


def _tc_encode(text_f, vis_f, W_t, b_t, W_v, b_v, W_g0):
    N, T = text_f.shape
    V = vis_f.shape[1]
    H = W_t.shape[1]
    RB = 2000
    assert N % RB == 0

    def body(t_ref, v_ref, wt_ref, bt_ref, wv_ref, bv_ref, wg_ref, o_ref):
        ht = jnp.maximum(
            jnp.dot(t_ref[...], wt_ref[...], preferred_element_type=jnp.float32)
            + bt_ref[...], 0.0)
        hv = jnp.maximum(
            jnp.dot(v_ref[...], wv_ref[...], preferred_element_type=jnp.float32)
            + bv_ref[...], 0.0)
        h = jnp.concatenate([ht, hv], axis=1)
        o_ref[...] = jnp.dot(h, wg_ref[...], preferred_element_type=jnp.float32)

    return pl.pallas_call(
        body,
        grid=(N // RB,),
        in_specs=[
            pl.BlockSpec((RB, T), lambda i: (i, 0)),
            pl.BlockSpec((RB, V), lambda i: (i, 0)),
            pl.BlockSpec((T, H), lambda i: (0, 0)),
            pl.BlockSpec((1, H), lambda i: (0, 0)),
            pl.BlockSpec((V, H), lambda i: (0, 0)),
            pl.BlockSpec((1, H), lambda i: (0, 0)),
            pl.BlockSpec((2 * H, H), lambda i: (0, 0)),
        ],
        out_specs=pl.BlockSpec((RB, H), lambda i: (i, 0)),
        out_shape=jax.ShapeDtypeStruct((N, H), jnp.float32),
    )(text_f, vis_f, W_t, b_t, W_v, b_v, W_g0)


def _tc_prescale(z, deg, N, n_pad):
    H = z.shape[1]

    def body(z_ref, dg_ref, o_ref, iv_ref):
        d = dg_ref[0] + dg_ref[1]
        inv = 1.0 / jnp.sqrt(jnp.maximum(d, 1.0))
        invt = jnp.transpose(inv, (1, 0))
        iv_ref[...] = invt
        o_ref[...] = z_ref[...] * invt[:N, 1:2]

    return pl.pallas_call(
        body,
        out_shape=(
            jax.ShapeDtypeStruct((N, H), jnp.float32),
            jax.ShapeDtypeStruct((n_pad, 2), jnp.float32),
        ),
    )(z, deg)


def _tc_mid(agg, invs, b_g0, W_g1, N):
    H = W_g1.shape[0]
    RB = 2000
    assert N % RB == 0

    def body(a_ref, iv_ref, b_ref, w_ref, o_ref):
        a = a_ref[:, :H] + a_ref[:, H:]
        iv = iv_ref[...]
        h = jnp.maximum(a * iv[:, 0:1] + b_ref[...], 0.0)
        z = jnp.dot(h, w_ref[...], preferred_element_type=jnp.float32)
        o_ref[...] = z * iv[:, 1:2]

    return pl.pallas_call(
        body,
        grid=(N // RB,),
        in_specs=[
            pl.BlockSpec((RB, NC * H), lambda i: (i, 0)),
            pl.BlockSpec((RB, 2), lambda i: (i, 0)),
            pl.BlockSpec((1, H), lambda i: (0, 0)),
            pl.BlockSpec((H, H), lambda i: (0, 0)),
        ],
        out_specs=pl.BlockSpec((RB, H), lambda i: (i, 0)),
        out_shape=jax.ShapeDtypeStruct((N, H), jnp.float32),
    )(agg, invs, b_g0, W_g1)


def _tc_head(agg, invs, b_g1, W_head, b_head, N):
    H, C = W_head.shape
    RB = 2000
    assert N % RB == 0

    def body(a_ref, iv_ref, b_ref, w_ref, bh_ref, o_ref):
        a = a_ref[:, :H] + a_ref[:, H:]
        iv = iv_ref[...]
        h = jnp.maximum(a * iv[:, 0:1] + b_ref[...], 0.0)
        o_ref[...] = (
            jnp.dot(h, w_ref[...], preferred_element_type=jnp.float32) + bh_ref[...]
        )

    return pl.pallas_call(
        body,
        grid=(N // RB,),
        in_specs=[
            pl.BlockSpec((RB, NC * H), lambda i: (i, 0)),
            pl.BlockSpec((RB, 2), lambda i: (i, 0)),
            pl.BlockSpec((1, H), lambda i: (0, 0)),
            pl.BlockSpec((H, C), lambda i: (0, 0)),
            pl.BlockSpec((1, C), lambda i: (0, 0)),
        ],
        out_specs=pl.BlockSpec((RB, C), lambda i: (i, 0)),
        out_shape=jax.ShapeDtypeStruct((N, C), jnp.float32),
    )(agg, invs, b_g1, W_head, b_head)


def kernel(edge_index, text_f, vis_f, W_t, b_t, W_v, b_v, W_g0, b_g0, W_g1, b_g1,
           W_head, b_head):
    N = text_f.shape[0]
    E = edge_index.shape[1]
    n_pad = -(-N // (NS * 64)) * (NS * 64)

    deg = _sc_degrees(edge_index, n_pad)
    z0 = _tc_encode(text_f, vis_f, W_t, b_t.reshape(1, -1), W_v,
                    b_v.reshape(1, -1), W_g0)
    p0, invs = _tc_prescale(z0, deg, N, n_pad)
    agg0 = _sc_aggregate(edge_index, p0, n_pad)
    p1 = _tc_mid(agg0, invs, b_g0.reshape(1, -1), W_g1, N)
    agg1 = _sc_aggregate(edge_index, p1, n_pad)
    out = _tc_head(agg1, invs, b_g1.reshape(1, -1), W_head, b_head.reshape(1, -1), N)
    return out


# speedup vs baseline: 1.2527x; 1.0011x over previous
import jax
import jax.numpy as jnp
from jax import lax
from jax.experimental import pallas as pl
from jax.experimental.pallas import tpu as pltpu
from jax.experimental.pallas import tpu_sc as plsc

NC = 2
NS = 16
NW = NC * NS


def _mesh():
    return plsc.VectorSubcoreMesh(
        core_axis_name="c", subcore_axis_name="s", num_cores=NC, num_subcores=NS
    )


def _fill(ref, n, value):
    def body(i, _):
        ref[pl.ds(i * 16, 16)] = jnp.full((16,), value, jnp.float32)
        return 0
    lax.fori_loop(0, n // 16, body, 0)


def _sc_degrees(eflat, n_pad):
    E = eflat.shape[1]
    CH = 128
    NB = 6
    assert E % NW == 0
    per_tile = E // NW
    full = per_tile // CH
    tail = per_tile % CH
    rounds, rem = divmod(full, NB)
    assert tail % 16 == 0 and per_tile % 8 == 0
    rpt = n_pad // NS

    def body(e_hbm, out_hbm, sbt, dbt, ones_v, onest_v, zer_v,
             din_sp, dout_sp, *rest):
        sbs = rest[0:NB]
        dbs = rest[NB:2 * NB]
        semA = rest[2 * NB:3 * NB]
        semB = rest[3 * NB:4 * NB]
        semS1 = rest[4 * NB:5 * NB]
        semS2 = rest[5 * NB:6 * NB]
        c = lax.axis_index("c")
        s = lax.axis_index("s")
        wid = c * NS + s
        base0 = wid * per_tile

        _fill(zer_v, rpt, 0.0)
        _fill(ones_v, CH, 1.0)
        if tail:
            _fill(onest_v, tail, 1.0)
        pltpu.sync_copy(zer_v, din_sp.at[pl.ds(s * rpt, rpt)])
        pltpu.sync_copy(zer_v, dout_sp.at[pl.ds(s * rpt, rpt)])
        plsc.subcore_barrier()

        bufs = tuple(zip(sbs, dbs, semA, semB, semS1, semS2))

        def pbody(i, _):
            descs = []
            for k, (sb, db, sA, sB, sS1, sS2) in enumerate(bufs):
                b0 = base0 + (NB * i + k) * CH

                @pl.when(i > 0)
                def _(sb=sb, db=db, sS1=sS1, sS2=sS2):
                    pltpu.make_async_copy(ones_v, dout_sp.at[sb], sS1).wait()
                    pltpu.make_async_copy(ones_v, din_sp.at[db], sS2).wait()
                descs.append((
                    pltpu.async_copy(e_hbm.at[0, pl.ds(b0, CH)], sb, sA),
                    pltpu.async_copy(e_hbm.at[1, pl.ds(b0, CH)], db, sB),
                ))
            for (dA, dB), (sb, db, _, _, sS1, sS2) in zip(descs, bufs):
                dA.wait()
                pltpu.async_copy(ones_v, dout_sp.at[sb], sS1, add=True)
                dB.wait()
                pltpu.async_copy(ones_v, din_sp.at[db], sS2, add=True)
            return 0

        lax.fori_loop(0, rounds, pbody, 0)
        if rounds > 0:
            for sb, db, _, _, sS1, sS2 in bufs:
                pltpu.make_async_copy(ones_v, dout_sp.at[sb], sS1).wait()
                pltpu.make_async_copy(ones_v, din_sp.at[db], sS2).wait()

        for r in range(rem):
            b0 = base0 + (rounds * NB + r) * CH
            sb, db = bufs[0][0], bufs[0][1]
            pltpu.sync_copy(e_hbm.at[0, pl.ds(b0, CH)], sb)
            pltpu.sync_copy(ones_v, dout_sp.at[sb], add=True)
            pltpu.sync_copy(e_hbm.at[1, pl.ds(b0, CH)], db)
            pltpu.sync_copy(ones_v, din_sp.at[db], add=True)

        if tail:
            bt = base0 + full * CH
            pltpu.sync_copy(e_hbm.at[0, pl.ds(bt, tail)], sbt)
            pltpu.sync_copy(onest_v, dout_sp.at[sbt], add=True)
            pltpu.sync_copy(e_hbm.at[1, pl.ds(bt, tail)], dbt)
            pltpu.sync_copy(onest_v, din_sp.at[dbt], add=True)

        plsc.subcore_barrier()

        pltpu.sync_copy(din_sp.at[pl.ds(s * rpt, rpt)],
                        out_hbm.at[c, 0, pl.ds(s * rpt, rpt)])
        pltpu.sync_copy(dout_sp.at[pl.ds(s * rpt, rpt)],
                        out_hbm.at[c, 1, pl.ds(s * rpt, rpt)])

    f = pl.kernel(
        body,
        out_type=jax.ShapeDtypeStruct((NC, 2, n_pad), jnp.float32),
        mesh=_mesh(),
        scratch_types=(
            [
                pltpu.VMEM((max(tail, 16),), jnp.int32),
                pltpu.VMEM((max(tail, 16),), jnp.int32),
                pltpu.VMEM((CH,), jnp.float32),
                pltpu.VMEM((max(tail, 16),), jnp.float32),
                pltpu.VMEM((rpt,), jnp.float32),
                pltpu.VMEM_SHARED((n_pad,), jnp.float32),
                pltpu.VMEM_SHARED((n_pad,), jnp.float32),
            ]
            + [pltpu.VMEM((CH,), jnp.int32)] * (2 * NB)
            + [pltpu.SemaphoreType.DMA] * (4 * NB)
        ),
        compiler_params=pltpu.CompilerParams(use_tc_tiling_on_sc=False),
    )
    return f(eflat)


def _sc_aggregate(eflat, p, n_pad):
    E = eflat.shape[1]
    H = p.shape[1]
    CH = 128
    NB = 6
    assert E % NW == 0
    per_tile = E // NW
    full = per_tile // CH
    tail = per_tile % CH
    rounds, rem = divmod(full, NB)
    assert tail % 8 == 0 and per_tile % 8 == 0
    rpt = n_pad // NS
    ZR = 64
    assert rpt % ZR == 0

    def body(e_hbm, p_hbm, out_hbm, sbig, dbt, rowst, zer_v, acc_sp, *rest):
        dbs = rest[0:NB]
        rows_l = rest[NB:2 * NB]
        semD = rest[2 * NB:3 * NB]
        semG = rest[3 * NB:4 * NB]
        semS = rest[4 * NB:5 * NB]
        c = lax.axis_index("c")
        s = lax.axis_index("s")
        wid = c * NS + s
        base0 = wid * per_tile

        pre = pltpu.async_copy(e_hbm.at[0, pl.ds(base0, per_tile)], sbig, semG[0])

        def zfill(i, _):
            zer_v[i, pl.ds(0, 16)] = jnp.zeros((16,), jnp.float32)
            zer_v[i, pl.ds(16, 16)] = jnp.zeros((16,), jnp.float32)
            zer_v[i, pl.ds(32, 16)] = jnp.zeros((16,), jnp.float32)
            zer_v[i, pl.ds(48, 16)] = jnp.zeros((16,), jnp.float32)
            return 0
        lax.fori_loop(0, ZR, zfill, 0)

        def zcopy(i, _):
            pltpu.sync_copy(zer_v, acc_sp.at[pl.ds(s * rpt + i * ZR, ZR)])
            return 0
        lax.fori_loop(0, rpt // ZR, zcopy, 0)
        pre.wait()
        plsc.subcore_barrier()

        bufs = tuple(zip(dbs, rows_l, semD, semG, semS))

        def tbody(i, _):
            descs = []
            for k, (db, rows, sD, sG, sS) in enumerate(bufs):
                b0 = (NB * i + k) * CH
                @pl.when(i > 0)
                def _(db=db, rows=rows, sS=sS):
                    pltpu.make_async_copy(rows, acc_sp.at[db], sS).wait()
                descs.append((
                    pltpu.async_copy(e_hbm.at[1, pl.ds(base0 + b0, CH)], db, sD),
                    pltpu.async_copy(p_hbm.at[sbig.at[pl.ds(b0, CH)]], rows, sG),
                ))
            for (d, g), (db, rows, _, _, sS) in zip(descs, bufs):
                d.wait()
                g.wait()
                pltpu.async_copy(rows, acc_sp.at[db], sS, add=True)
            return 0

        lax.fori_loop(0, rounds, tbody, 0)
        if rounds > 0:
            for db, rows, _, _, sS in bufs:
                pltpu.make_async_copy(rows, acc_sp.at[db], sS).wait()

        for r in range(rem):
            b0 = (rounds * NB + r) * CH
            db, rows, sD, sG, sS = bufs[0]
            pltpu.sync_copy(e_hbm.at[1, pl.ds(base0 + b0, CH)], db)
            pltpu.async_copy(p_hbm.at[sbig.at[pl.ds(b0, CH)]], rows, sG).wait()
            pltpu.sync_copy(rows, acc_sp.at[db], add=True)

        if tail:
            bt = full * CH
            pltpu.sync_copy(e_hbm.at[1, pl.ds(base0 + bt, tail)], dbt)
            pltpu.async_copy(p_hbm.at[sbig.at[pl.ds(bt, tail)]], rowst,
                             semG[0]).wait()
            pltpu.sync_copy(rowst, acc_sp.at[dbt], add=True)

        plsc.subcore_barrier()
        pltpu.sync_copy(acc_sp.at[pl.ds(s * rpt, rpt)],
                        out_hbm.at[pl.ds(s * rpt, rpt), pl.ds(c * H, H)])

    f = pl.kernel(
        body,
        out_type=jax.ShapeDtypeStruct((n_pad, NC * H), jnp.float32),
        mesh=_mesh(),
        scratch_types=(
            [
                pltpu.VMEM((per_tile,), jnp.int32),
                pltpu.VMEM((max(tail, 8),), jnp.int32),
                pltpu.VMEM((max(tail, 8), H), jnp.float32),
                pltpu.VMEM((ZR, H), jnp.float32),
                pltpu.VMEM_SHARED((n_pad, H), jnp.float32),
            ]
            + [pltpu.VMEM((CH,), jnp.int32)] * NB
            + [pltpu.VMEM((CH, H), jnp.float32)] * NB
            + [pltpu.SemaphoreType.DMA] * (3 * NB)
        ),
        compiler_params=pltpu.CompilerParams(use_tc_tiling_on_sc=False),
    )
    return f(eflat, p)


def _tc_encode(text_f, vis_f, W_t, b_t, W_v, b_v, W_g0):
    N, T = text_f.shape
    V = vis_f.shape[1]
    H = W_t.shape[1]
    RB = 2000
    assert N % RB == 0

    def body(t_ref, v_ref, wt_ref, bt_ref, wv_ref, bv_ref, wg_ref, o_ref):
        ht = jnp.maximum(
            jnp.dot(t_ref[...], wt_ref[...], preferred_element_type=jnp.float32)
            + bt_ref[...], 0.0)
        hv = jnp.maximum(
            jnp.dot(v_ref[...], wv_ref[...], preferred_element_type=jnp.float32)
            + bv_ref[...], 0.0)
        h = jnp.concatenate([ht, hv], axis=1)
        o_ref[...] = jnp.dot(h, wg_ref[...], preferred_element_type=jnp.float32)

    return pl.pallas_call(
        body,
        grid=(N // RB,),
        in_specs=[
            pl.BlockSpec((RB, T), lambda i: (i, 0)),
            pl.BlockSpec((RB, V), lambda i: (i, 0)),
            pl.BlockSpec((T, H), lambda i: (0, 0)),
            pl.BlockSpec((1, H), lambda i: (0, 0)),
            pl.BlockSpec((V, H), lambda i: (0, 0)),
            pl.BlockSpec((1, H), lambda i: (0, 0)),
            pl.BlockSpec((2 * H, H), lambda i: (0, 0)),
        ],
        out_specs=pl.BlockSpec((RB, H), lambda i: (i, 0)),
        out_shape=jax.ShapeDtypeStruct((N, H), jnp.float32),
    )(text_f, vis_f, W_t, b_t, W_v, b_v, W_g0)


def _tc_prescale(z, deg, N, n_pad):
    H = z.shape[1]

    def body(z_ref, dg_ref, o_ref, iv_ref):
        d = dg_ref[0] + dg_ref[1]
        inv = 1.0 / jnp.sqrt(jnp.maximum(d, 1.0))
        invt = jnp.transpose(inv, (1, 0))
        iv_ref[...] = invt
        o_ref[...] = z_ref[...] * invt[:N, 1:2]

    return pl.pallas_call(
        body,
        out_shape=(
            jax.ShapeDtypeStruct((N, H), jnp.float32),
            jax.ShapeDtypeStruct((n_pad, 2), jnp.float32),
        ),
    )(z, deg)


def _tc_mid(agg, invs, b_g0, W_g1, N):
    H = W_g1.shape[0]
    RB = 2000
    assert N % RB == 0

    def body(a_ref, iv_ref, b_ref, w_ref, o_ref):
        a = a_ref[:, :H] + a_ref[:, H:]
        iv = iv_ref[...]
        h = jnp.maximum(a * iv[:, 0:1] + b_ref[...], 0.0)
        z = jnp.dot(h, w_ref[...], preferred_element_type=jnp.float32)
        o_ref[...] = z * iv[:, 1:2]

    return pl.pallas_call(
        body,
        grid=(N // RB,),
        in_specs=[
            pl.BlockSpec((RB, NC * H), lambda i: (i, 0)),
            pl.BlockSpec((RB, 2), lambda i: (i, 0)),
            pl.BlockSpec((1, H), lambda i: (0, 0)),
            pl.BlockSpec((H, H), lambda i: (0, 0)),
        ],
        out_specs=pl.BlockSpec((RB, H), lambda i: (i, 0)),
        out_shape=jax.ShapeDtypeStruct((N, H), jnp.float32),
    )(agg, invs, b_g0, W_g1)


def _tc_head(agg, invs, b_g1, W_head, b_head, N):
    H, C = W_head.shape
    RB = 2000
    assert N % RB == 0

    def body(a_ref, iv_ref, b_ref, w_ref, bh_ref, o_ref):
        a = a_ref[:, :H] + a_ref[:, H:]
        iv = iv_ref[...]
        h = jnp.maximum(a * iv[:, 0:1] + b_ref[...], 0.0)
        o_ref[...] = (
            jnp.dot(h, w_ref[...], preferred_element_type=jnp.float32) + bh_ref[...]
        )

    return pl.pallas_call(
        body,
        grid=(N // RB,),
        in_specs=[
            pl.BlockSpec((RB, NC * H), lambda i: (i, 0)),
            pl.BlockSpec((RB, 2), lambda i: (i, 0)),
            pl.BlockSpec((1, H), lambda i: (0, 0)),
            pl.BlockSpec((H, C), lambda i: (0, 0)),
            pl.BlockSpec((1, C), lambda i: (0, 0)),
        ],
        out_specs=pl.BlockSpec((RB, C), lambda i: (i, 0)),
        out_shape=jax.ShapeDtypeStruct((N, C), jnp.float32),
    )(agg, invs, b_g1, W_head, b_head)


def kernel(edge_index, text_f, vis_f, W_t, b_t, W_v, b_v, W_g0, b_g0, W_g1, b_g1,
           W_head, b_head):
    N = text_f.shape[0]
    E = edge_index.shape[1]
    n_pad = -(-N // (NS * 64)) * (NS * 64)

    deg = _sc_degrees(edge_index, n_pad)
    z0 = _tc_encode(text_f, vis_f, W_t, b_t.reshape(1, -1), W_v,
                    b_v.reshape(1, -1), W_g0)
    p0, invs = _tc_prescale(z0, deg, N, n_pad)
    agg0 = _sc_aggregate(edge_index, p0, n_pad)
    p1 = _tc_mid(agg0, invs, b_g0.reshape(1, -1), W_g1, N)
    agg1 = _sc_aggregate(edge_index, p1, n_pad)
    out = _tc_head(agg1, invs, b_g1.reshape(1, -1), W_head, b_head.reshape(1, -1), N)
    return out
